# baseline (speedup 1.0000x reference)
import functools

import jax
import jax.numpy as jnp
from jax import lax
from jax.experimental import pallas as pl
from jax.experimental.pallas import tpu as pltpu
from jax.experimental.pallas import tpu_sc as plsc

B, L, D = 2, 512, 128
MAXPOS = 31
NBIN = 63
P = B * L * L
NW = 32
PER_W = P // NW
C = 128
NCH = PER_W // C
NBUF = 4


def _table_body(w_emb_ref, w_chain_ref, w_proj_ref, t_ref):
    wp1 = w_proj_ref[0:D, :]
    wp2 = w_proj_ref[D:2 * D, :]
    t1 = jnp.dot(w_emb_ref[...], wp1, preferred_element_type=jnp.float32)
    t2 = jnp.dot(w_chain_ref[...], wp2, preferred_element_type=jnp.float32)
    is_inter = lax.broadcasted_iota(jnp.int32, (NBIN + 1, 1), 0) == NBIN
    t_ref[...] = t1 + jnp.where(is_inter, t2[0:1, :], t2[1:2, :])


def _make_sc_gather():
    mesh = plsc.VectorSubcoreMesh(core_axis_name="c", subcore_axis_name="s")

    @functools.partial(
        pl.kernel,
        mesh=mesh,
        out_type=jax.ShapeDtypeStruct((P, D), jnp.float32),
        scratch_types=[
            pltpu.VMEM((NBIN + 1, D), jnp.float32),
            pltpu.VMEM_SHARED((NBIN + 1, D), jnp.float32),
            pltpu.VMEM((B * L,), jnp.int32),
            pltpu.VMEM((PER_W,), jnp.int32),
        ]
        + [pltpu.VMEM((C,), jnp.int32)] * NBUF
        + [pltpu.VMEM((C, D), jnp.float32)] * NBUF
        + [pltpu.SemaphoreType.DMA] * (2 * NBUF + 2),
        compiler_params=pltpu.CompilerParams(needs_layout_passes=False),
    )
    def gather(table_hbm, idx_hbm, sc_hbm, out_hbm, table_v, table_sh,
               idx_v, scm, *bufs):
        seqb = bufs[:NBUF]
        rows = bufs[NBUF:2 * NBUF]
        gsem = bufs[2 * NBUF:3 * NBUF]
        ssem = bufs[3 * NBUF:4 * NBUF]
        isem = bufs[4 * NBUF]
        jsem = bufs[4 * NBUF + 1]
        wid = lax.axis_index("s") * 2 + lax.axis_index("c")
        base = wid * PER_W

        def compute_indices(g, slot):
            r0 = base + g * C
            bi = r0 // L
            jbase = (bi // L) * L + (r0 % L)
            splat_i = plsc.load_gather(idx_v, [jnp.full((16,), bi, jnp.int32)])
            for q in range(C // 16):
                jv = idx_v[pl.ds(jbase + 16 * q, 16)]
                s = jnp.clip(jv - splat_i + MAXPOS, 0, NBIN - 1)
                scv = scm[pl.ds(g * C + 16 * q, 16)]
                seqb[slot][pl.ds(16 * q, 16)] = NBIN + scv * (s - NBIN)

        def start_gather(g, slot):
            compute_indices(g, slot)
            pltpu.async_copy(table_sh.at[seqb[slot]], rows[slot], gsem[slot])

        def wait_gather(g, slot):
            pltpu.make_async_copy(table_sh.at[seqb[slot]], rows[slot],
                                  gsem[slot]).wait()

        def start_scatter(g, slot):
            pltpu.async_copy(rows[slot],
                             out_hbm.at[pl.ds(base + g * C, C)], ssem[slot])

        def wait_scatter(g, slot):
            pltpu.make_async_copy(rows[slot],
                                  out_hbm.at[pl.ds(base + g * C, C)],
                                  ssem[slot]).wait()

        pltpu.async_copy(sc_hbm.at[pl.ds(base, PER_W)], scm, isem)
        pltpu.async_copy(idx_hbm, idx_v, jsem)

        @pl.when(lax.axis_index("s") == 0)
        def _():
            pltpu.sync_copy(table_hbm, table_v)
            pltpu.sync_copy(table_v, table_sh)

        plsc.subcore_barrier()
        pltpu.make_async_copy(sc_hbm.at[pl.ds(base, PER_W)], scm, isem).wait()
        pltpu.make_async_copy(idx_hbm, idx_v, jsem).wait()
        for s in range(NBUF - 1):
            start_gather(s, s)

        def body(h, carry):
            for j in range(NBUF):
                g = h * NBUF + j
                wait_gather(g, j)
                start_scatter(g, j)
                nslot = (j + NBUF - 1) % NBUF
                nxt = g + NBUF - 1

                @pl.when(nxt < NCH)
                def _():
                    @pl.when(g >= 1)
                    def _():
                        wait_scatter(g - 1, nslot)
                    start_gather(nxt, nslot)

            return carry

        lax.fori_loop(0, NCH // NBUF, body, 0)
        for s in range(NBUF):
            wait_scatter(NCH - NBUF + s, s)

    return gather


_sc_gather = _make_sc_gather()


def kernel(idx, same_chain, W_emb, W_chain, W_proj):
    table = pl.pallas_call(
        _table_body,
        out_shape=jax.ShapeDtypeStruct((NBIN + 1, D), jnp.float32),
    )(W_emb, W_chain, W_proj)

    out = _sc_gather(table, idx.reshape(B * L), same_chain.reshape(P))
    return out.reshape(B, L, L, D)


# speedup vs baseline: 1.0073x; 1.0073x over previous
import functools

import jax
import jax.numpy as jnp
from jax import lax
from jax.experimental import pallas as pl
from jax.experimental.pallas import tpu as pltpu
from jax.experimental.pallas import tpu_sc as plsc

B, L, D = 2, 512, 128
MAXPOS = 31
NBIN = 63
P = B * L * L
NW = 32
PER_W = P // NW
C = 128
NCH = PER_W // C


def _prologue_body(idx_row_ref, idx_col_ref, sc_ref, w_emb_ref, w_chain_ref,
                   w_proj_ref, seq_ref, t_ref):
    row = idx_row_ref[0]
    col = idx_col_ref[0]
    s = jnp.clip(row - col + MAXPOS, 0, NBIN - 1)
    sc = sc_ref[0]
    seq_ref[0] = s * sc + NBIN * (1 - sc)

    @pl.when(pl.program_id(0) == 0)
    def _():
        wp1 = w_proj_ref[0:D, :]
        wp2 = w_proj_ref[D:2 * D, :]
        t1 = jnp.dot(w_emb_ref[...], wp1, preferred_element_type=jnp.float32)
        t2 = jnp.dot(w_chain_ref[...], wp2,
                     preferred_element_type=jnp.float32)
        is_inter = lax.broadcasted_iota(jnp.int32, (NBIN + 1, 1), 0) == NBIN
        t_ref[...] = t1 + jnp.where(is_inter, t2[0:1, :], t2[1:2, :])


NBUF = 4


def _make_sc_gather():
    mesh = plsc.VectorSubcoreMesh(core_axis_name="c", subcore_axis_name="s")

    @functools.partial(
        pl.kernel,
        mesh=mesh,
        out_type=jax.ShapeDtypeStruct((P, D), jnp.float32),
        scratch_types=[
            pltpu.VMEM((NBIN + 1, D), jnp.float32),
            pltpu.VMEM_SHARED((NBIN + 1, D), jnp.float32),
            pltpu.VMEM((PER_W,), jnp.int32),
        ]
        + [pltpu.VMEM((C, D), jnp.float32)] * NBUF
        + [pltpu.SemaphoreType.DMA] * (2 * NBUF + 1),
        compiler_params=pltpu.CompilerParams(needs_layout_passes=False),
    )
    def gather(table_hbm, seq_hbm, out_hbm, table_v, table_sh, idx_all, *bufs):
        rows = bufs[:NBUF]
        gsem = bufs[NBUF:2 * NBUF]
        ssem = bufs[2 * NBUF:3 * NBUF]
        isem = bufs[3 * NBUF]
        wid = lax.axis_index("s") * 2 + lax.axis_index("c")
        base = wid * PER_W

        def start_gather(g, slot):
            pltpu.async_copy(table_sh.at[idx_all.at[pl.ds(g * C, C)]],
                             rows[slot], gsem[slot])

        def wait_gather(g, slot):
            pltpu.make_async_copy(table_sh.at[idx_all.at[pl.ds(g * C, C)]],
                                  rows[slot], gsem[slot]).wait()

        def start_scatter(g, slot):
            pltpu.async_copy(rows[slot],
                             out_hbm.at[pl.ds(base + g * C, C)], ssem[slot])

        def wait_scatter(g, slot):
            pltpu.make_async_copy(rows[slot],
                                  out_hbm.at[pl.ds(base + g * C, C)],
                                  ssem[slot]).wait()

        pltpu.async_copy(seq_hbm.at[pl.ds(base, PER_W)], idx_all, isem)

        @pl.when(lax.axis_index("s") == 0)
        def _():
            pltpu.sync_copy(table_hbm, table_v)
            pltpu.sync_copy(table_v, table_sh)

        plsc.subcore_barrier()
        pltpu.make_async_copy(seq_hbm.at[pl.ds(base, PER_W)], idx_all,
                              isem).wait()
        for s in range(NBUF - 1):
            start_gather(s, s)

        def body(h, carry):
            for j in range(NBUF):
                g = h * NBUF + j
                wait_gather(g, j)
                start_scatter(g, j)
                nslot = (j + NBUF - 1) % NBUF
                nxt = g + NBUF - 1

                @pl.when(nxt < NCH)
                def _():
                    @pl.when(g >= 1)
                    def _():
                        wait_scatter(g - 1, nslot)
                    start_gather(nxt, nslot)

            return carry

        lax.fori_loop(0, NCH // NBUF, body, 0)
        for s in range(NBUF):
            wait_scatter(NCH - NBUF + s, s)

    return gather


_sc_gather = _make_sc_gather()


def kernel(idx, same_chain, W_emb, W_chain, W_proj):
    idx_row = idx.reshape(B, 1, L)
    idx_col = idx.reshape(B, L, 1)
    seqsep, table = pl.pallas_call(
        _prologue_body,
        grid=(B,),
        in_specs=[
            pl.BlockSpec((1, 1, L), lambda b: (b, 0, 0)),
            pl.BlockSpec((1, L, 1), lambda b: (b, 0, 0)),
            pl.BlockSpec((1, L, L), lambda b: (b, 0, 0)),
            pl.BlockSpec((NBIN + 1, D), lambda b: (0, 0)),
            pl.BlockSpec((2, D), lambda b: (0, 0)),
            pl.BlockSpec((2 * D, D), lambda b: (0, 0)),
        ],
        out_specs=[
            pl.BlockSpec((1, L, L), lambda b: (b, 0, 0)),
            pl.BlockSpec((NBIN + 1, D), lambda b: (0, 0)),
        ],
        out_shape=[
            jax.ShapeDtypeStruct((B, L, L), jnp.int32),
            jax.ShapeDtypeStruct((NBIN + 1, D), jnp.float32),
        ],
    )(idx_row, idx_col, same_chain, W_emb, W_chain, W_proj)

    out = _sc_gather(table, seqsep.reshape(P))
    return out.reshape(B, L, L, D)
